# baseline (speedup 1.0000x reference)
import jax
import jax.numpy as jnp
from jax.experimental import pallas as pl
from jax.experimental.pallas import tpu as pltpu

_BN = 16384
_KC = 256
_SUB = 8


def _loss_kernel(et_hbm, c_ref, out_ref, ebuf, caug_ref, sems):
    i = pl.program_id(0)
    n_i = pl.num_programs(0)

    def in_dma(slot, idx):
        return pltpu.make_async_copy(
            et_hbm.at[:, pl.ds(idx * _BN, _BN)],
            ebuf.at[slot],
            sems.at[slot])

    @pl.when(i == 0)
    def _prologue():
        in_dma(0, 0).start()
        c = c_ref[...]
        c_sq = jnp.sum(c * c, axis=1, keepdims=True)
        hi = c_sq.astype(jnp.bfloat16)
        lo = (c_sq - hi.astype(jnp.float32)).astype(jnp.bfloat16)
        caug_ref[...] = jnp.concatenate(
            [(-2.0 * c).astype(jnp.bfloat16), hi, lo], axis=1)
        out_ref[...] = jnp.zeros_like(out_ref)

    @pl.when(i + 1 < n_i)
    def _prefetch():
        in_dma((i + 1) % 2, i + 1).start()

    in_dma(i % 2, i).wait()
    et = ebuf[i % 2]
    bn = et.shape[1]
    et_aug = jnp.concatenate(
        [et.astype(jnp.bfloat16),
         jnp.ones((2, bn), jnp.bfloat16)], axis=0)
    c_aug = caug_ref[...]
    k = c_aug.shape[0]

    m_acc = None
    for j in range(k // _KC):
        cj = c_aug[j * _KC:(j + 1) * _KC, :]
        pj = jax.lax.dot_general(
            cj, et_aug, (((1,), (0,)), ((), ())),
            preferred_element_type=jnp.float32)
        tiles = [pj[t * _SUB:(t + 1) * _SUB, :] for t in range(_KC // _SUB)]
        while len(tiles) > 1:
            tiles = [jnp.minimum(tiles[t], tiles[t + 1])
                     for t in range(0, len(tiles) - 1, 2)] + (
                         [tiles[-1]] if len(tiles) % 2 else [])
        mj = tiles[0]
        m_acc = mj if m_acc is None else jnp.minimum(m_acc, mj)
    m_col = jnp.min(m_acc, axis=0, keepdims=True)
    e_sq = jnp.sum(et * et, axis=0, keepdims=True)
    partial = jnp.sum(m_col + e_sq).reshape(1, 1)
    out_ref[...] += partial


def kernel(embedded, centers):
    n, d = embedded.shape
    k = centers.shape[0]
    et = embedded.T
    et = pltpu.with_memory_space_constraint(et, pltpu.MemorySpace.HBM)
    grid = n // _BN
    total = pl.pallas_call(
        _loss_kernel,
        grid=(grid,),
        in_specs=[
            pl.BlockSpec(memory_space=pl.ANY),
            pl.BlockSpec((k, d), lambda i: (0, 0)),
        ],
        out_specs=pl.BlockSpec((1, 1), lambda i: (0, 0)),
        out_shape=jax.ShapeDtypeStruct((1, 1), jnp.float32),
        scratch_shapes=[
            pltpu.VMEM((2, d, _BN), jnp.float32),
            pltpu.VMEM((k, d + 2), jnp.bfloat16),
            pltpu.SemaphoreType.DMA((2,)),
        ],
    )(et, centers)
    return total[0, 0] / n


# speedup vs baseline: 1.0488x; 1.0488x over previous
import jax
import jax.numpy as jnp
from jax.experimental import pallas as pl
from jax.experimental.pallas import tpu as pltpu

_BN = 16384
_KC = 256
_SUB = 8


def _loss_kernel(et_hbm, c_ref, out_ref, ebuf, caug_ref, sems):
    i = pl.program_id(0)
    n_i = pl.num_programs(0)

    def in_dma(slot, idx):
        return pltpu.make_async_copy(
            et_hbm.at[:, pl.ds(idx * _BN, _BN)],
            ebuf.at[slot],
            sems.at[slot])

    @pl.when(i == 0)
    def _prologue():
        in_dma(0, 0).start()
        ct = c_ref[...]
        c_sq = jnp.sum(ct * ct, axis=0, keepdims=True)
        hi = c_sq.astype(jnp.bfloat16)
        lo = (c_sq - hi.astype(jnp.float32)).astype(jnp.bfloat16)
        caug_ref[...] = jnp.concatenate(
            [(-2.0 * ct).astype(jnp.bfloat16), hi, lo], axis=0)
        out_ref[...] = jnp.zeros_like(out_ref)

    @pl.when(i + 1 < n_i)
    def _prefetch():
        in_dma((i + 1) % 2, i + 1).start()

    in_dma(i % 2, i).wait()
    et = ebuf[i % 2]
    bn = et.shape[1]
    et_aug = jnp.concatenate(
        [et.astype(jnp.bfloat16),
         jnp.ones((2, bn), jnp.bfloat16)], axis=0)
    c_aug = caug_ref[...]
    k = c_aug.shape[1]

    m_acc = None
    for j in range(k // _KC):
        cj = c_aug[:, j * _KC:(j + 1) * _KC]
        pj = jax.lax.dot_general(
            cj, et_aug, (((0,), (0,)), ((), ())),
            preferred_element_type=jnp.float32)
        tiles = [pj[t * _SUB:(t + 1) * _SUB, :] for t in range(_KC // _SUB)]
        while len(tiles) > 1:
            tiles = [jnp.minimum(tiles[t], tiles[t + 1])
                     for t in range(0, len(tiles) - 1, 2)] + (
                         [tiles[-1]] if len(tiles) % 2 else [])
        mj = tiles[0]
        m_acc = mj if m_acc is None else jnp.minimum(m_acc, mj)
    m_col = jnp.min(m_acc, axis=0, keepdims=True)
    e_sq = jnp.sum(et * et, axis=0, keepdims=True)
    partial = jnp.sum(m_col + e_sq).reshape(1, 1)
    out_ref[...] += partial


def kernel(embedded, centers):
    n, d = embedded.shape
    k = centers.shape[0]
    et = embedded.T
    ct = centers.T
    et = pltpu.with_memory_space_constraint(et, pltpu.MemorySpace.HBM)
    grid = n // _BN
    total = pl.pallas_call(
        _loss_kernel,
        grid=(grid,),
        in_specs=[
            pl.BlockSpec(memory_space=pl.ANY),
            pl.BlockSpec((d, k), lambda i: (0, 0)),
        ],
        out_specs=pl.BlockSpec((1, 1), lambda i: (0, 0)),
        out_shape=jax.ShapeDtypeStruct((1, 1), jnp.float32),
        scratch_shapes=[
            pltpu.VMEM((2, d, _BN), jnp.float32),
            pltpu.VMEM((d + 2, k), jnp.bfloat16),
            pltpu.SemaphoreType.DMA((2,)),
        ],
    )(et, ct)
    return total[0, 0] / n
